# baseline (speedup 1.0000x reference)
import jax
import jax.numpy as jnp
from jax.experimental import pallas as pl
from jax.experimental.pallas import tpu as pltpu

_B, _N, _NODE_DIM, _EDGE_DIM, _N_LAYERS = 32, 100, 64, 16, 4
_E = _N * _N
_G = 8


def _silu(x):
    a = 0.5 * x
    t = jnp.tanh(a)
    return a * t + a


def _silu_b(x, bh):
    a = 0.5 * x + bh
    t = jnp.tanh(a)
    return a * t + a


def _mm(a, b):
    return jax.lax.dot_general(
        a, b, (((1,), (0,)), ((), ())), preferred_element_type=jnp.float32
    )


def _gnn_kernel(raw_ref, RT2_ref, Rtm_ref,
                P_ref, niblk_ref,
                Aaug_ref, Bk_ref, Wck_ref, W2k_ref, b2h_ref,
                U1a_ref, U1b_ref, U2_ref, ub2h_ref,
                Attk_ref, abh_ref, out_ref):
    RT2 = RT2_ref[...]
    Rtm = Rtm_ref[...]

    ws8 = raw_ref[1] * (1.0 / 28.0)
    wd8 = raw_ref[0] * (jnp.pi / 180.0)
    yw8 = raw_ref[2] * (jnp.pi / 180.0)
    lx8 = raw_ref[3] * 2.0 - 1.0
    ly8 = raw_ref[4] * 2.0 - 1.0
    wx8 = ws8 * jnp.cos(wd8)
    wy8 = ws8 * jnp.sin(wd8)

    z48 = jnp.zeros((6 * _G, _N), jnp.float32)
    z16 = jnp.zeros((2 * _G, _N), jnp.float32)
    S_all = jnp.concatenate([ws8, yw8, lx8, ly8, wx8, wy8], axis=0)
    D_all = jnp.concatenate([lx8, ly8], axis=0)
    L = jnp.concatenate([
        jnp.concatenate([S_all, z48], axis=1),
        jnp.concatenate([z16, D_all], axis=1),
    ], axis=0)
    SD = _mm(L, RT2)

    dx = SD[48:56] - SD[16:24]
    dy = SD[56:64] - SD[24:32]
    radial = jnp.sqrt(dx * dx + dy * dy)
    wdot = SD[32:40] * dx + SD[40:48] * dy
    wcross = SD[32:40] * dy - SD[40:48] * dx
    onesE = jnp.ones((1, _E), jnp.float32)
    ef = jnp.concatenate(
        [radial, SD[0:8], wdot, wcross, SD[8:16], onesE], axis=0)

    e = _mm(P_ref[...], ef)
    seg = _mm(ef[0:40], Rtm)
    ones100 = jnp.ones((1, _N), jnp.float32)
    nf = jnp.concatenate([ws8, seg, ones100], axis=0)
    h = _mm(niblk_ref[...], nf)

    Attk = Attk_ref[...]
    abh = abh_ref[0, 0]
    for i in range(_N_LAYERS):
        h_aug = jnp.concatenate([h, ones100], axis=0)
        hs = _mm(Aaug_ref[i], h_aug)
        hd = _mm(Bk_ref[i], h)
        X2 = jnp.concatenate([hs, hd], axis=1)
        pre = _mm(X2, RT2) + _mm(Wck_ref[i], e)
        m = _silu(pre)
        m = _silu_b(_mm(W2k_ref[i], m), b2h_ref[i])
        att = _mm(Attk, m)
        s = 0.5 * jnp.tanh(0.5 * att + abh) + 0.5
        m = m * s
        aggr = _mm(m, Rtm)
        u = _silu(_mm(U1a_ref[i], h_aug) + _mm(U1b_ref[i], aggr))
        u = _silu_b(_mm(U2_ref[i], u), ub2h_ref[i])
        h = h + u
        if i < _N_LAYERS - 1:
            e = e + m
    out_ref[...] = h.reshape(_NODE_DIM, _G, _N)


def kernel(wind_direction, wind_speed, yaw, layout, node_in_W, node_in_b,
           edge_in_W, edge_in_b, msg_W1, msg_b1, msg_W2, msg_b2,
           upd_W1, upd_b1, upd_W2, upd_b2, att_W, att_b, edge_index):
    f32 = jnp.float32
    raw = jnp.concatenate(
        [wind_direction, wind_speed, yaw, layout], axis=-1
    ).transpose(2, 0, 1).astype(f32)

    ii = jax.lax.broadcasted_iota(jnp.int32, (_N, _E), 0)
    ll = jax.lax.broadcasted_iota(jnp.int32, (_N, _E), 1)
    R = (ll // _N == ii).astype(f32)
    T = (ll % _N == ii).astype(f32)
    RT2 = jnp.concatenate([R, T], axis=0)
    le = jax.lax.broadcasted_iota(jnp.int32, (_E, _N), 0)
    ie = jax.lax.broadcasted_iota(jnp.int32, (_E, _N), 1)
    Rtm = ((le // _N == ie) & (le // _N != le % _N)).astype(f32)

    I8 = jnp.eye(_G, dtype=f32)

    def kron8(W):
        return jnp.kron(W, I8)

    def rep8(b):
        return jnp.repeat(b.reshape(-1), _G).reshape(-1, 1)

    niWt = node_in_W.T
    eiWt = edge_in_W.T
    mW1t = msg_W1.transpose(0, 2, 1)
    mW2t = msg_W2.transpose(0, 2, 1)
    uW1t = upd_W1.transpose(0, 2, 1)
    uW2t = upd_W2.transpose(0, 2, 1)

    P = jnp.concatenate([kron8(eiWt), rep8(edge_in_b)], axis=1)
    niblk = jnp.concatenate([kron8(niWt), rep8(node_in_b)], axis=1)
    Aaug = jnp.stack([
        jnp.concatenate([kron8(mW1t[i][:, 0:64]), rep8(msg_b1[i])], axis=1)
        for i in range(_N_LAYERS)])
    Bk = jnp.stack([kron8(mW1t[i][:, 64:128]) for i in range(_N_LAYERS)])
    Wck = jnp.stack([kron8(mW1t[i][:, 128:144]) for i in range(_N_LAYERS)])
    W2k = jnp.stack([kron8(mW2t[i]) for i in range(_N_LAYERS)])
    b2h = jnp.stack([0.5 * rep8(msg_b2[i]) for i in range(_N_LAYERS)])
    U1a = jnp.stack([
        jnp.concatenate([kron8(uW1t[i][:, 0:64]), rep8(upd_b1[i])], axis=1)
        for i in range(_N_LAYERS)])
    U1b = jnp.stack([kron8(uW1t[i][:, 64:80]) for i in range(_N_LAYERS)])
    U2 = jnp.stack([kron8(uW2t[i]) for i in range(_N_LAYERS)])
    ub2h = jnp.stack([0.5 * rep8(upd_b2[i]) for i in range(_N_LAYERS)])
    Attk = kron8(jnp.tile(att_W.T, (_EDGE_DIM, 1)))
    abh = (0.5 * att_b).reshape(1, 1)

    def full(x):
        return pl.BlockSpec(x.shape, lambda b: (0,) * x.ndim)

    out = pl.pallas_call(
        _gnn_kernel,
        grid=(_B // _G,),
        in_specs=[
            pl.BlockSpec((5, _G, _N), lambda b: (0, b, 0)),
            full(RT2), full(Rtm), full(P), full(niblk),
            full(Aaug), full(Bk), full(Wck), full(W2k), full(b2h),
            full(U1a), full(U1b), full(U2), full(ub2h),
            full(Attk), full(abh),
        ],
        out_specs=pl.BlockSpec((_NODE_DIM, _G, _N), lambda b: (0, b, 0)),
        out_shape=jax.ShapeDtypeStruct((_NODE_DIM, _B, _N), f32),
        compiler_params=pltpu.CompilerParams(
            dimension_semantics=("arbitrary",),
        ),
    )(raw, RT2, Rtm, P, niblk, Aaug, Bk, Wck, W2k, b2h,
      U1a, U1b, U2, ub2h, Attk, abh)
    return out.transpose(1, 2, 0)


# speedup vs baseline: 2.6977x; 2.6977x over previous
import jax
import jax.numpy as jnp
from jax.experimental import pallas as pl
from jax.experimental.pallas import tpu as pltpu

_B, _N, _NODE_DIM, _EDGE_DIM, _N_LAYERS = 32, 100, 64, 16, 4
_E = _N * _N
_G = 8
_f32 = jnp.float32


def _silu(x):
    a = 0.5 * x
    t = jnp.tanh(a)
    return a * t + a


def _silu_b(x, bh):
    a = 0.5 * x + bh
    t = jnp.tanh(a)
    return a * t + a


def _mm(a, b):
    return jax.lax.dot_general(
        a, b, (((1,), (0,)), ((), ())), preferred_element_type=_f32
    )


def _dgT(a, b):
    return jax.lax.dot_general(
        a, b, (((0,), (0,)), ((), ())), preferred_element_type=_f32
    )


def _iota2(shape, dim):
    return jax.lax.broadcasted_iota(jnp.int32, shape, dim)


def _gnn_kernel(wd_ref, ws_ref, yaw_ref, lx_ref, ly_ref,
                RT2_ref, Rtm_ref,
                niW_ref, nib_ref, eiW_ref, eib_ref,
                mW1_ref, mb1_ref, mW2_ref, mb2_ref,
                uW1_ref, ub1_ref, uW2_ref, ub2_ref,
                aW_ref, ab_ref, out_ref,
                P_s, ni_s, Aaug_s, Bk_s, Wck_s, W2k_s, b2h_s,
                U1a_s, U1b_s, U2_s, ub2h_s, Attk_s):

    @pl.when(pl.program_id(0) == 0)
    def _build():
        M8 = (_iota2((512, 512), 0) % _G == _iota2((512, 512), 1) % _G
              ).astype(_f32)
        F64 = (_iota2((512, 64), 0) // _G == _iota2((512, 64), 1)
               ).astype(_f32)
        F16 = F64[0:128, 0:16]
        E64 = (_iota2((64, 512), 0) == _iota2((64, 512), 1) // _G
               ).astype(_f32)
        E16 = E64[0:16, 0:128]
        E6 = E64[0:6, 0:48]
        E5 = E64[0:5, 0:40]

        def blk(W, Fo, Er, M):
            return _mm(Fo, _dgT(W, Er)) * M

        P_s[...] = jnp.concatenate(
            [blk(eiW_ref[...], F16, E5, M8[0:128, 0:40]),
             _mm(F16, eib_ref[...])], axis=1)
        ni_s[...] = jnp.concatenate(
            [blk(niW_ref[...], F64, E6, M8[0:512, 0:48]),
             _mm(F64, nib_ref[...])], axis=1)
        Attk_s[...] = jnp.broadcast_to(
            _dgT(aW_ref[...], E16), (128, 128)) * M8[0:128, 0:128]
        for i in range(_N_LAYERS):
            W1 = mW1_ref[i]
            Aaug_s[i] = jnp.concatenate(
                [blk(W1[0:64], F16, E64, M8[0:128, 0:512]),
                 _mm(F16, mb1_ref[i])], axis=1)
            Bk_s[i] = blk(W1[64:128], F16, E64, M8[0:128, 0:512])
            Wck_s[i] = blk(W1[128:144], F16, E16, M8[0:128, 0:128])
            W2k_s[i] = blk(mW2_ref[i], F16, E16, M8[0:128, 0:128])
            b2h_s[i] = 0.5 * _mm(F16, mb2_ref[i])
            U1 = uW1_ref[i]
            U1a_s[i] = jnp.concatenate(
                [blk(U1[0:64], F64, E64, M8),
                 _mm(F64, ub1_ref[i])], axis=1)
            U1b_s[i] = blk(U1[64:80], F64, E16, M8[0:512, 0:128])
            U2_s[i] = blk(uW2_ref[i], F64, E64, M8)
            ub2h_s[i] = 0.5 * _mm(F64, ub2_ref[i])

    RT2 = RT2_ref[...]
    Rtm = Rtm_ref[...]

    ws8 = ws_ref[...] * (1.0 / 28.0)
    wd8 = wd_ref[...] * (jnp.pi / 180.0)
    yw8 = yaw_ref[...] * (jnp.pi / 180.0)
    lx8 = lx_ref[...] * 2.0 - 1.0
    ly8 = ly_ref[...] * 2.0 - 1.0
    wx8 = ws8 * jnp.cos(wd8)
    wy8 = ws8 * jnp.sin(wd8)

    z48 = jnp.zeros((6 * _G, _N), _f32)
    z16 = jnp.zeros((2 * _G, _N), _f32)
    S_all = jnp.concatenate([ws8, yw8, lx8, ly8, wx8, wy8], axis=0)
    D_all = jnp.concatenate([lx8, ly8], axis=0)
    L = jnp.concatenate([
        jnp.concatenate([S_all, z48], axis=1),
        jnp.concatenate([z16, D_all], axis=1),
    ], axis=0)
    SD = _mm(L, RT2)

    dx = SD[48:56] - SD[16:24]
    dy = SD[56:64] - SD[24:32]
    radial = jnp.sqrt(dx * dx + dy * dy)
    wdot = SD[32:40] * dx + SD[40:48] * dy
    wcross = SD[32:40] * dy - SD[40:48] * dx
    onesE = jnp.ones((1, _E), _f32)
    ef = jnp.concatenate(
        [radial, SD[0:8], wdot, wcross, SD[8:16], onesE], axis=0)

    e = _mm(P_s[...], ef)
    seg = _mm(ef[0:40], Rtm)
    ones100 = jnp.ones((1, _N), _f32)
    nf = jnp.concatenate([ws8, seg, ones100], axis=0)
    h = _mm(ni_s[...], nf)

    Attk = Attk_s[...]
    abh = 0.5 * ab_ref[0, 0]
    for i in range(_N_LAYERS):
        h_aug = jnp.concatenate([h, ones100], axis=0)
        hs = _mm(Aaug_s[i], h_aug)
        hd = _mm(Bk_s[i], h)
        X2 = jnp.concatenate([hs, hd], axis=1)
        pre = _mm(X2, RT2) + _mm(Wck_s[i], e)
        m = _silu(pre)
        m = _silu_b(_mm(W2k_s[i], m), b2h_s[i])
        att = _mm(Attk, m)
        s = 0.5 * jnp.tanh(0.5 * att + abh) + 0.5
        m = m * s
        aggr = _mm(m, Rtm)
        u = _silu(_mm(U1a_s[i], h_aug) + _mm(U1b_s[i], aggr))
        u = _silu_b(_mm(U2_s[i], u), ub2h_s[i])
        h = h + u
        if i < _N_LAYERS - 1:
            e = e + m
    out_ref[...] = h.reshape(_NODE_DIM, _G, _N)


def kernel(wind_direction, wind_speed, yaw, layout, node_in_W, node_in_b,
           edge_in_W, edge_in_b, msg_W1, msg_b1, msg_W2, msg_b2,
           upd_W1, upd_b1, upd_W2, upd_b2, att_W, att_b, edge_index):
    wd = wind_direction.reshape(_B, _N).astype(_f32)
    ws = wind_speed.reshape(_B, _N).astype(_f32)
    yw = yaw.reshape(_B, _N).astype(_f32)
    lx = layout[:, :, 0].astype(_f32)
    ly = layout[:, :, 1].astype(_f32)

    ii = _iota2((_N, _E), 0)
    ll = _iota2((_N, _E), 1)
    R = (ll // _N == ii).astype(_f32)
    T = (ll % _N == ii).astype(_f32)
    RT2 = jnp.concatenate([R, T], axis=0)
    le = _iota2((_E, _N), 0)
    ie = _iota2((_E, _N), 1)
    Rtm = ((le // _N == ie) & (le // _N != le % _N)).astype(_f32)

    def full(x):
        return pl.BlockSpec(x.shape, lambda b: (0,) * x.ndim)

    def batched(x):
        return pl.BlockSpec((_G, _N), lambda b: (b, 0))

    ins = [
        wd, ws, yw, lx, ly, RT2, Rtm,
        node_in_W.astype(_f32), node_in_b.reshape(_NODE_DIM, 1).astype(_f32),
        edge_in_W.astype(_f32), edge_in_b.reshape(_EDGE_DIM, 1).astype(_f32),
        msg_W1.astype(_f32), msg_b1.reshape(_N_LAYERS, _EDGE_DIM, 1).astype(_f32),
        msg_W2.astype(_f32), msg_b2.reshape(_N_LAYERS, _EDGE_DIM, 1).astype(_f32),
        upd_W1.astype(_f32), upd_b1.reshape(_N_LAYERS, _NODE_DIM, 1).astype(_f32),
        upd_W2.astype(_f32), upd_b2.reshape(_N_LAYERS, _NODE_DIM, 1).astype(_f32),
        att_W.astype(_f32), att_b.reshape(1, 1).astype(_f32),
    ]
    in_specs = [batched(wd), batched(ws), batched(yw), batched(lx), batched(ly),
                full(RT2), full(Rtm)] + [full(x) for x in ins[7:]]

    scratch = [
        pltpu.VMEM((128, 41), _f32),
        pltpu.VMEM((512, 49), _f32),
        pltpu.VMEM((_N_LAYERS, 128, 513), _f32),
        pltpu.VMEM((_N_LAYERS, 128, 512), _f32),
        pltpu.VMEM((_N_LAYERS, 128, 128), _f32),
        pltpu.VMEM((_N_LAYERS, 128, 128), _f32),
        pltpu.VMEM((_N_LAYERS, 128, 1), _f32),
        pltpu.VMEM((_N_LAYERS, 512, 513), _f32),
        pltpu.VMEM((_N_LAYERS, 512, 128), _f32),
        pltpu.VMEM((_N_LAYERS, 512, 512), _f32),
        pltpu.VMEM((_N_LAYERS, 512, 1), _f32),
        pltpu.VMEM((128, 128), _f32),
    ]

    out = pl.pallas_call(
        _gnn_kernel,
        grid=(_B // _G,),
        in_specs=in_specs,
        out_specs=pl.BlockSpec((_NODE_DIM, _G, _N), lambda b: (0, b, 0)),
        out_shape=jax.ShapeDtypeStruct((_NODE_DIM, _B, _N), _f32),
        scratch_shapes=scratch,
        compiler_params=pltpu.CompilerParams(
            dimension_semantics=("arbitrary",),
        ),
    )(*ins)
    return out.transpose(1, 2, 0)


# speedup vs baseline: 2.8726x; 1.0648x over previous
import jax
import jax.numpy as jnp
import numpy as np
from jax.experimental import pallas as pl
from jax.experimental.pallas import tpu as pltpu

_B, _N, _NODE_DIM, _EDGE_DIM, _N_LAYERS = 32, 100, 64, 16, 4
_E = _N * _N
_G = 8
_f32 = jnp.float32

_lanes = np.arange(_E)
_rows = np.arange(_N)[:, None]
_RT2_NP = np.concatenate(
    [_lanes // _N == _rows, _lanes % _N == _rows], axis=0).astype(np.float32)
_RTM_NP = ((_lanes[:, None] // _N == _rows.T)
           & (_lanes[:, None] // _N != _lanes[:, None] % _N)
           ).astype(np.float32)


def _silu(x):
    a = 0.5 * x
    t = jnp.tanh(a)
    return a * t + a


def _silu_b(x, bh):
    a = 0.5 * x + bh
    t = jnp.tanh(a)
    return a * t + a


def _mm(a, b):
    return jax.lax.dot_general(
        a, b, (((1,), (0,)), ((), ())), preferred_element_type=_f32
    )


def _dgT(a, b):
    return jax.lax.dot_general(
        a, b, (((0,), (0,)), ((), ())), preferred_element_type=_f32
    )


def _iota2(shape, dim):
    return jax.lax.broadcasted_iota(jnp.int32, shape, dim)


def _gnn_kernel(wd_ref, ws_ref, yaw_ref, lx_ref, ly_ref,
                RT2_ref, Rtm_ref,
                niW_ref, nib_ref, eiW_ref, eib_ref,
                mW1_ref, mb1_ref, mW2_ref, mb2_ref,
                uW1_ref, ub1_ref, uW2_ref, ub2_ref,
                aW_ref, ab_ref, out_ref,
                P_s, ni_s, Aaug_s, Bk_s, Wck_s, W2k_s, b2h_s,
                U1a_s, U1b_s, U2_s, ub2h_s, Attk_s):

    @pl.when(pl.program_id(0) == 0)
    def _build():
        M8 = (_iota2((512, 512), 0) % _G == _iota2((512, 512), 1) % _G
              ).astype(_f32)
        F64 = (_iota2((512, 64), 0) // _G == _iota2((512, 64), 1)
               ).astype(_f32)
        F16 = F64[0:128, 0:16]
        E64 = (_iota2((64, 512), 0) == _iota2((64, 512), 1) // _G
               ).astype(_f32)
        E16 = E64[0:16, 0:128]
        E6 = E64[0:6, 0:48]
        E5 = E64[0:5, 0:40]

        def blk(W, Fo, Er, M):
            return _mm(Fo, _dgT(W, Er)) * M

        P_s[...] = jnp.concatenate(
            [blk(eiW_ref[...], F16, E5, M8[0:128, 0:40]),
             _mm(F16, eib_ref[...])], axis=1)
        ni_s[...] = jnp.concatenate(
            [blk(niW_ref[...], F64, E6, M8[0:512, 0:48]),
             _mm(F64, nib_ref[...])], axis=1)
        Attk_s[...] = jnp.broadcast_to(
            _dgT(aW_ref[...], E16), (128, 128)) * M8[0:128, 0:128]
        for i in range(_N_LAYERS):
            W1 = mW1_ref[i]
            Aaug_s[i] = jnp.concatenate(
                [blk(W1[0:64], F16, E64, M8[0:128, 0:512]),
                 _mm(F16, mb1_ref[i])], axis=1)
            Bk_s[i] = blk(W1[64:128], F16, E64, M8[0:128, 0:512])
            Wck_s[i] = blk(W1[128:144], F16, E16, M8[0:128, 0:128])
            W2k_s[i] = blk(mW2_ref[i], F16, E16, M8[0:128, 0:128])
            b2h_s[i] = 0.5 * _mm(F16, mb2_ref[i])
            U1 = uW1_ref[i]
            U1a_s[i] = jnp.concatenate(
                [blk(U1[0:64], F64, E64, M8),
                 _mm(F64, ub1_ref[i])], axis=1)
            U1b_s[i] = blk(U1[64:80], F64, E16, M8[0:512, 0:128])
            U2_s[i] = blk(uW2_ref[i], F64, E64, M8)
            ub2h_s[i] = 0.5 * _mm(F64, ub2_ref[i])

    RT2 = RT2_ref[...]
    Rtm = Rtm_ref[...]

    ws8 = ws_ref[...] * (1.0 / 28.0)
    wd8 = wd_ref[...] * (jnp.pi / 180.0)
    yw8 = yaw_ref[...] * (jnp.pi / 180.0)
    lx8 = lx_ref[...] * 2.0 - 1.0
    ly8 = ly_ref[...] * 2.0 - 1.0
    wx8 = ws8 * jnp.cos(wd8)
    wy8 = ws8 * jnp.sin(wd8)

    z48 = jnp.zeros((6 * _G, _N), _f32)
    z16 = jnp.zeros((2 * _G, _N), _f32)
    S_all = jnp.concatenate([ws8, yw8, lx8, ly8, wx8, wy8], axis=0)
    D_all = jnp.concatenate([lx8, ly8], axis=0)
    L = jnp.concatenate([
        jnp.concatenate([S_all, z48], axis=1),
        jnp.concatenate([z16, D_all], axis=1),
    ], axis=0)
    SD = _mm(L, RT2)

    dx = SD[48:56] - SD[16:24]
    dy = SD[56:64] - SD[24:32]
    radial = jnp.sqrt(dx * dx + dy * dy)
    wdot = SD[32:40] * dx + SD[40:48] * dy
    wcross = SD[32:40] * dy - SD[40:48] * dx
    onesE = jnp.ones((1, _E), _f32)
    ef = jnp.concatenate(
        [radial, SD[0:8], wdot, wcross, SD[8:16], onesE], axis=0)

    e = _mm(P_s[...], ef)
    seg = _mm(ef[0:40], Rtm)
    ones100 = jnp.ones((1, _N), _f32)
    nf = jnp.concatenate([ws8, seg, ones100], axis=0)
    h = _mm(ni_s[...], nf)

    Attk = Attk_s[...]
    abh = 0.5 * ab_ref[0, 0]
    for i in range(_N_LAYERS):
        h_aug = jnp.concatenate([h, ones100], axis=0)
        hs = _mm(Aaug_s[i], h_aug)
        hd = _mm(Bk_s[i], h)
        X2 = jnp.concatenate([hs, hd], axis=1)
        pre = _mm(X2, RT2) + _mm(Wck_s[i], e)
        m = _silu(pre)
        m = _silu_b(_mm(W2k_s[i], m), b2h_s[i])
        att = _mm(Attk, m)
        s = 0.5 * jnp.tanh(0.5 * att + abh) + 0.5
        m = m * s
        aggr = _mm(m, Rtm)
        u = _silu(_mm(U1a_s[i], h_aug) + _mm(U1b_s[i], aggr))
        u = _silu_b(_mm(U2_s[i], u), ub2h_s[i])
        h = h + u
        if i < _N_LAYERS - 1:
            e = e + m
    out_ref[...] = h.reshape(_NODE_DIM, _G, _N)


def kernel(wind_direction, wind_speed, yaw, layout, node_in_W, node_in_b,
           edge_in_W, edge_in_b, msg_W1, msg_b1, msg_W2, msg_b2,
           upd_W1, upd_b1, upd_W2, upd_b2, att_W, att_b, edge_index):
    wd = wind_direction.reshape(_B, _N).astype(_f32)
    ws = wind_speed.reshape(_B, _N).astype(_f32)
    yw = yaw.reshape(_B, _N).astype(_f32)
    lx = layout[:, :, 0].astype(_f32)
    ly = layout[:, :, 1].astype(_f32)

    def full(x):
        return pl.BlockSpec(x.shape, lambda b: (0,) * x.ndim)

    def batched(x):
        return pl.BlockSpec((_G, _N), lambda b: (b, 0))

    RT2 = jnp.asarray(_RT2_NP)
    Rtm = jnp.asarray(_RTM_NP)
    ins = [
        wd, ws, yw, lx, ly, RT2, Rtm,
        node_in_W.astype(_f32), node_in_b.reshape(_NODE_DIM, 1).astype(_f32),
        edge_in_W.astype(_f32), edge_in_b.reshape(_EDGE_DIM, 1).astype(_f32),
        msg_W1.astype(_f32), msg_b1.reshape(_N_LAYERS, _EDGE_DIM, 1).astype(_f32),
        msg_W2.astype(_f32), msg_b2.reshape(_N_LAYERS, _EDGE_DIM, 1).astype(_f32),
        upd_W1.astype(_f32), upd_b1.reshape(_N_LAYERS, _NODE_DIM, 1).astype(_f32),
        upd_W2.astype(_f32), upd_b2.reshape(_N_LAYERS, _NODE_DIM, 1).astype(_f32),
        att_W.astype(_f32), att_b.reshape(1, 1).astype(_f32),
    ]
    in_specs = [batched(wd), batched(ws), batched(yw), batched(lx), batched(ly),
                full(RT2), full(Rtm)] + [full(x) for x in ins[7:]]

    scratch = [
        pltpu.VMEM((128, 41), _f32),
        pltpu.VMEM((512, 49), _f32),
        pltpu.VMEM((_N_LAYERS, 128, 513), _f32),
        pltpu.VMEM((_N_LAYERS, 128, 512), _f32),
        pltpu.VMEM((_N_LAYERS, 128, 128), _f32),
        pltpu.VMEM((_N_LAYERS, 128, 128), _f32),
        pltpu.VMEM((_N_LAYERS, 128, 1), _f32),
        pltpu.VMEM((_N_LAYERS, 512, 513), _f32),
        pltpu.VMEM((_N_LAYERS, 512, 128), _f32),
        pltpu.VMEM((_N_LAYERS, 512, 512), _f32),
        pltpu.VMEM((_N_LAYERS, 512, 1), _f32),
        pltpu.VMEM((128, 128), _f32),
    ]

    out = pl.pallas_call(
        _gnn_kernel,
        grid=(_B // _G,),
        in_specs=in_specs,
        out_specs=pl.BlockSpec((_NODE_DIM, _G, _N), lambda b: (0, b, 0)),
        out_shape=jax.ShapeDtypeStruct((_NODE_DIM, _B, _N), _f32),
        scratch_shapes=scratch,
        compiler_params=pltpu.CompilerParams(
            dimension_semantics=("arbitrary",),
        ),
    )(*ins)
    return out.transpose(1, 2, 0)
